# baseline (speedup 1.0000x reference)
import functools

import jax
import jax.numpy as jnp
from jax import lax
from jax.experimental import pallas as pl
from jax.experimental.pallas import tpu as pltpu
from jax.experimental.pallas import tpu_sc as plsc

_LANES = 16
_NBUF = 2
_BBLK = 512
_SUB = 128


@functools.lru_cache(maxsize=None)
def _build_sc_kernel(b_total: int, s_len: int, n_ctx: int, d: int,
                     nc: int, ns: int):
    nw = nc * ns
    bpb = b_total // _BBLK
    items = s_len * bpb
    items_pw = items // nw
    d_vecs = d // _LANES
    dt_n = d // 8
    bt_n = _BBLK // _SUB

    mesh = plsc.VectorSubcoreMesh(core_axis_name="c", subcore_axis_name="s")

    @functools.partial(
        pl.kernel,
        out_type=jax.ShapeDtypeStruct((s_len, dt_n, b_total // _SUB, 8, _SUB),
                                      jnp.float32),
        mesh=mesh,
        scratch_types=[
            pltpu.VMEM((n_ctx, d), jnp.float32),
            pltpu.VMEM((2, d), jnp.float32),
            pltpu.VMEM((2 * n_ctx * d,), jnp.float32),
            [pltpu.VMEM((_BBLK,), jnp.int32)] * _NBUF,
            [pltpu.VMEM((_BBLK,), jnp.int32)] * _NBUF,
            [pltpu.VMEM((dt_n, bt_n, 8, _SUB), jnp.float32)] * _NBUF,
            [pltpu.SemaphoreType.DMA] * _NBUF,
            [pltpu.SemaphoreType.DMA] * _NBUF,
        ],
        compiler_params=pltpu.CompilerParams(use_tc_tiling_on_sc=False,
                                             needs_layout_passes=False),
    )
    def sc_kernel(pos_hbm, seq_hbm, ptab_hbm, stab_hbm, out_hbm,
                  ptab_v, stab_v, comb, posv, seqv, outv, sem_in, sem_w):
        c = lax.axis_index("c")
        s = lax.axis_index("s")
        wid = c * ns + s
        item0 = wid * items_pw

        pltpu.sync_copy(ptab_hbm, ptab_v)
        pltpu.sync_copy(stab_hbm, stab_v)

        def row(p, carry):
            for dc in range(d_vecs):
                sl = pl.ds(dc * _LANES, _LANES)
                v = ptab_v[p, sl]
                comb[pl.ds(p * d + dc * _LANES, _LANES)] = v + stab_v[0, sl]
                comb[pl.ds(n_ctx * d + p * d + dc * _LANES, _LANES)] = (
                    v + stab_v[1, sl])
            return carry

        lax.fori_loop(0, n_ctx, row, 0)

        def load_idx(item, b):
            t = item // bpb
            b0 = (item % bpb) * _BBLK
            pltpu.async_copy(pos_hbm.at[t, pl.ds(b0, _BBLK)], posv[b],
                             sem_in[b])
            pltpu.async_copy(seq_hbm.at[t, pl.ds(b0, _BBLK)], seqv[b],
                             sem_in[b])

        def drain_idx(item, b):
            t = item // bpb
            b0 = (item % bpb) * _BBLK
            pltpu.make_async_copy(pos_hbm.at[t, pl.ds(b0, _BBLK)], posv[b],
                                  sem_in[b]).wait()
            pltpu.make_async_copy(seq_hbm.at[t, pl.ds(b0, _BBLK)], seqv[b],
                                  sem_in[b]).wait()

        def out_slice(item):
            t = item // bpb
            bb = item % bpb
            return out_hbm.at[t, :, pl.ds(bb * bt_n, bt_n), :, :]

        for b in range(_NBUF):
            load_idx(item0 + b, b)

        def super_body(m2, carry):
            for b in range(_NBUF):
                m = m2 * _NBUF + b
                item = item0 + m
                drain_idx(item, b)

                @pl.when(m2 > 0)
                def _drain_wb():
                    pltpu.make_async_copy(outv[b], out_slice(item),
                                          sem_w[b]).wait()

                for bt in range(bt_n):
                    for bcg in range(_SUB // _LANES):
                        sl = pl.ds(bt * _SUB + bcg * _LANES, _LANES)
                        ci = (seqv[b][sl] * (n_ctx * d)
                              + posv[b][sl] * d)
                        osl = pl.ds(bcg * _LANES, _LANES)

                        def dstep(dt, carry2, ci=ci, bt=bt, osl=osl):
                            base = ci + dt * 8
                            for dr in range(8):
                                v = plsc.load_gather(comb, [base + dr])
                                outv[b][dt, bt, dr, osl] = v
                            return carry2

                        lax.fori_loop(0, dt_n, dstep, 0)

                @pl.when(m + _NBUF < items_pw)
                def _prefetch():
                    load_idx(item + _NBUF, b)

                pltpu.async_copy(outv[b], out_slice(item), sem_w[b])
            return carry

        lax.fori_loop(0, items_pw // _NBUF, super_body, 0)

        for b in range(_NBUF):
            pltpu.make_async_copy(
                outv[b], out_slice(item0 + items_pw - _NBUF + b),
                sem_w[b]).wait()

    return sc_kernel


def kernel(positions, sequence_ids, pos_table, seq_table):
    b, s = positions.shape
    n_ctx, d = pos_table.shape

    info = plsc.get_sparse_core_info()
    nc, ns = info.num_cores, info.num_subcores
    nw = nc * ns

    assert (s * (b // _BBLK)) % (nw * _NBUF) == 0
    assert b % _BBLK == 0 and d % _LANES == 0

    sc = _build_sc_kernel(b, s, n_ctx, d, nc, ns)
    out5 = sc(positions.T.astype(jnp.int32), sequence_ids.T.astype(jnp.int32),
              pos_table.astype(jnp.float32), seq_table.astype(jnp.float32))
    return jnp.transpose(out5, (2, 4, 0, 1, 3)).reshape(b, s, d)


# speedup vs baseline: 1.0069x; 1.0069x over previous
import functools

import jax
import jax.numpy as jnp
from jax import lax
from jax.experimental import pallas as pl
from jax.experimental.pallas import tpu as pltpu
from jax.experimental.pallas import tpu_sc as plsc

_LANES = 16
_NBUF = 2
_BBLK = 512
_SUB = 128


@functools.lru_cache(maxsize=None)
def _build_sc_kernel(b_total: int, s_len: int, n_ctx: int, d: int,
                     nc: int, ns: int):
    nw = nc * ns
    bpb = b_total // _BBLK
    items = s_len * bpb
    items_pw = items // nw
    d_vecs = d // _LANES
    dt_n = d // 8
    bt_n = _BBLK // _SUB

    mesh = plsc.VectorSubcoreMesh(core_axis_name="c", subcore_axis_name="s")

    @functools.partial(
        pl.kernel,
        out_type=jax.ShapeDtypeStruct((s_len, dt_n, b_total // _SUB, 8, _SUB),
                                      jnp.float32),
        mesh=mesh,
        scratch_types=[
            pltpu.VMEM((n_ctx, d), jnp.float32),
            pltpu.VMEM((2, d), jnp.float32),
            pltpu.VMEM((2 * n_ctx * d,), jnp.float32),
            [pltpu.VMEM((_BBLK,), jnp.int32)] * _NBUF,
            [pltpu.VMEM((_BBLK,), jnp.int32)] * _NBUF,
            [pltpu.VMEM((dt_n, bt_n, 8, _SUB), jnp.float32)] * _NBUF,
            [pltpu.SemaphoreType.DMA] * _NBUF,
            [pltpu.SemaphoreType.DMA] * _NBUF,
        ],
        compiler_params=pltpu.CompilerParams(use_tc_tiling_on_sc=False,
                                             needs_layout_passes=False),
    )
    def sc_kernel(pos_hbm, seq_hbm, ptab_hbm, stab_hbm, out_hbm,
                  ptab_v, stab_v, comb, posv, seqv, outv, sem_in, sem_w):
        c = lax.axis_index("c")
        s = lax.axis_index("s")
        wid = c * ns + s
        item0 = wid * items_pw

        pltpu.sync_copy(ptab_hbm, ptab_v)
        pltpu.sync_copy(stab_hbm, stab_v)

        def row(p, carry):
            for dc in range(d_vecs):
                sl = pl.ds(dc * _LANES, _LANES)
                v = ptab_v[p, sl]
                comb[pl.ds(p * d + dc * _LANES, _LANES)] = v + stab_v[0, sl]
                comb[pl.ds(n_ctx * d + p * d + dc * _LANES, _LANES)] = (
                    v + stab_v[1, sl])
            return carry

        lax.fori_loop(0, n_ctx, row, 0)

        def load_idx(item, b):
            t = item // bpb
            b0 = (item % bpb) * _BBLK
            pltpu.async_copy(pos_hbm.at[t, pl.ds(b0, _BBLK)], posv[b],
                             sem_in[b])
            pltpu.async_copy(seq_hbm.at[t, pl.ds(b0, _BBLK)], seqv[b],
                             sem_in[b])

        def drain_idx(item, b):
            t = item // bpb
            b0 = (item % bpb) * _BBLK
            pltpu.make_async_copy(pos_hbm.at[t, pl.ds(b0, _BBLK)], posv[b],
                                  sem_in[b]).wait()
            pltpu.make_async_copy(seq_hbm.at[t, pl.ds(b0, _BBLK)], seqv[b],
                                  sem_in[b]).wait()

        def out_slice(item):
            t = item // bpb
            bb = item % bpb
            return out_hbm.at[t, :, pl.ds(bb * bt_n, bt_n), :, :]

        for b in range(_NBUF):
            load_idx(item0 + b, b)

        def super_body(m2, carry):
            for b in range(_NBUF):
                m = m2 * _NBUF + b
                item = item0 + m
                drain_idx(item, b)

                @pl.when(m2 > 0)
                def _drain_wb():
                    pltpu.make_async_copy(outv[b], out_slice(item),
                                          sem_w[b]).wait()

                def jgroup(j, carry2):
                    bt = j // (_SUB // _LANES)
                    bcg = j % (_SUB // _LANES)
                    sl = pl.ds(j * _LANES, _LANES)
                    ci = seqv[b][sl] * (n_ctx * d) + posv[b][sl] * d
                    osl = pl.ds(bcg * _LANES, _LANES)
                    for dd in range(d):
                        v = plsc.load_gather(comb, [ci + dd])
                        outv[b][dd // 8, bt, dd % 8, osl] = v
                    return carry2

                lax.fori_loop(0, _BBLK // _LANES, jgroup, 0)

                @pl.when(m + _NBUF < items_pw)
                def _prefetch():
                    load_idx(item + _NBUF, b)

                pltpu.async_copy(outv[b], out_slice(item), sem_w[b])
            return carry

        lax.fori_loop(0, items_pw // _NBUF, super_body, 0)

        for b in range(_NBUF):
            pltpu.make_async_copy(
                outv[b], out_slice(item0 + items_pw - _NBUF + b),
                sem_w[b]).wait()

    return sc_kernel


def kernel(positions, sequence_ids, pos_table, seq_table):
    b, s = positions.shape
    n_ctx, d = pos_table.shape

    info = plsc.get_sparse_core_info()
    nc, ns = info.num_cores, info.num_subcores
    nw = nc * ns

    assert (s * (b // _BBLK)) % (nw * _NBUF) == 0
    assert b % _BBLK == 0 and d % _LANES == 0

    sc = _build_sc_kernel(b, s, n_ctx, d, nc, ns)
    out5 = sc(positions.T.astype(jnp.int32), sequence_ids.T.astype(jnp.int32),
              pos_table.astype(jnp.float32), seq_table.astype(jnp.float32))
    return jnp.transpose(out5, (2, 4, 0, 1, 3)).reshape(b, s, d)


# speedup vs baseline: 1.7856x; 1.7734x over previous
import functools

import jax
import jax.numpy as jnp
from jax import lax
from jax.experimental import pallas as pl
from jax.experimental.pallas import tpu as pltpu
from jax.experimental.pallas import tpu_sc as plsc

_LANES = 16
_NBUF = 2
_BBLK = 512
_SUB = 128


@functools.lru_cache(maxsize=None)
def _build_sc_kernel(b_total: int, s_len: int, n_ctx: int, d: int,
                     nc: int, ns: int):
    nw = nc * ns
    bpb = b_total // _BBLK
    items = s_len * bpb
    items_pw = items // nw
    d_vecs = d // _LANES
    dt_n = d // 8
    bt_n = _BBLK // _SUB

    mesh = plsc.VectorSubcoreMesh(core_axis_name="c", subcore_axis_name="s")

    @functools.partial(
        pl.kernel,
        out_type=jax.ShapeDtypeStruct((s_len, dt_n, b_total // _SUB, 8, _SUB),
                                      jnp.float32),
        mesh=mesh,
        scratch_types=[
            pltpu.VMEM((n_ctx, d), jnp.float32),
            pltpu.VMEM((2, d), jnp.float32),
            pltpu.VMEM((2 * n_ctx * d,), jnp.float32),
            [pltpu.VMEM((_BBLK,), jnp.int32)] * _NBUF,
            [pltpu.VMEM((_BBLK,), jnp.int32)] * _NBUF,
            [pltpu.VMEM((dt_n, bt_n, 8, _SUB), jnp.float32)] * _NBUF,
            [pltpu.SemaphoreType.DMA] * _NBUF,
            [pltpu.SemaphoreType.DMA] * _NBUF,
        ],
        compiler_params=pltpu.CompilerParams(use_tc_tiling_on_sc=False,
                                             needs_layout_passes=False),
    )
    def sc_kernel(pos_hbm, seq_hbm, ptab_hbm, stab_hbm, out_hbm,
                  ptab_v, stab_v, comb, posv, seqv, outv, sem_in, sem_w):
        c = lax.axis_index("c")
        s = lax.axis_index("s")
        wid = c * ns + s
        item0 = wid * items_pw

        pltpu.sync_copy(ptab_hbm, ptab_v)
        pltpu.sync_copy(stab_hbm, stab_v)

        def row(p, carry):
            for dc in range(d_vecs):
                sl = pl.ds(dc * _LANES, _LANES)
                v = ptab_v[p, sl]
                comb[pl.ds(p * d + dc * _LANES, _LANES)] = v + stab_v[0, sl]
                comb[pl.ds(n_ctx * d + p * d + dc * _LANES, _LANES)] = (
                    v + stab_v[1, sl])
            return carry

        lax.fori_loop(0, n_ctx, row, 0)

        def load_idx(item, b):
            t = item // bpb
            b0 = (item % bpb) * _BBLK
            pltpu.async_copy(pos_hbm.at[t, pl.ds(b0, _BBLK)], posv[b],
                             sem_in[b])
            pltpu.async_copy(seq_hbm.at[t, pl.ds(b0, _BBLK)], seqv[b],
                             sem_in[b])

        def drain_idx(item, b):
            t = item // bpb
            b0 = (item % bpb) * _BBLK
            pltpu.make_async_copy(pos_hbm.at[t, pl.ds(b0, _BBLK)], posv[b],
                                  sem_in[b]).wait()
            pltpu.make_async_copy(seq_hbm.at[t, pl.ds(b0, _BBLK)], seqv[b],
                                  sem_in[b]).wait()

        def out_slice(item):
            t = item // bpb
            bb = item % bpb
            return out_hbm.at[t, :, pl.ds(bb * bt_n, bt_n), :, :]

        for b in range(_NBUF):
            load_idx(item0 + b, b)

        def super_body(m2, carry):
            for b in range(_NBUF):
                m = m2 * _NBUF + b
                item = item0 + m
                drain_idx(item, b)

                @pl.when(m2 > 0)
                def _drain_wb():
                    pltpu.make_async_copy(outv[b], out_slice(item),
                                          sem_w[b]).wait()

                @plsc.parallel_loop(0, _BBLK // _LANES)
                def jgroup(j):
                    bt = j // (_SUB // _LANES)
                    bcg = j % (_SUB // _LANES)
                    sl = pl.ds(j * _LANES, _LANES)
                    ci = seqv[b][sl] * (n_ctx * d) + posv[b][sl] * d
                    osl = pl.ds(bcg * _LANES, _LANES)
                    for dt in range(d // 8):
                        vs = [plsc.load_gather(comb, [ci + dt * 8 + dr])
                              for dr in range(8)]
                        for dr in range(8):
                            outv[b][dt, bt, dr, osl] = vs[dr]

                @pl.when(m + _NBUF < items_pw)
                def _prefetch():
                    load_idx(item + _NBUF, b)

                pltpu.async_copy(outv[b], out_slice(item), sem_w[b])
            return carry

        lax.fori_loop(0, items_pw // _NBUF, super_body, 0)

        for b in range(_NBUF):
            pltpu.make_async_copy(
                outv[b], out_slice(item0 + items_pw - _NBUF + b),
                sem_w[b]).wait()

    return sc_kernel


def kernel(positions, sequence_ids, pos_table, seq_table):
    b, s = positions.shape
    n_ctx, d = pos_table.shape

    info = plsc.get_sparse_core_info()
    nc, ns = info.num_cores, info.num_subcores
    nw = nc * ns

    assert (s * (b // _BBLK)) % (nw * _NBUF) == 0
    assert b % _BBLK == 0 and d % _LANES == 0

    sc = _build_sc_kernel(b, s, n_ctx, d, nc, ns)
    out5 = sc(positions.T.astype(jnp.int32), sequence_ids.T.astype(jnp.int32),
              pos_table.astype(jnp.float32), seq_table.astype(jnp.float32))
    return jnp.transpose(out5, (2, 4, 0, 1, 3)).reshape(b, s, d)


# speedup vs baseline: 7.4364x; 4.1646x over previous
import functools

import jax
import jax.numpy as jnp
from jax import lax
from jax.experimental import pallas as pl
from jax.experimental.pallas import tpu as pltpu
from jax.experimental.pallas import tpu_sc as plsc

_LANES = 16
_NBUF = 2
_BBLK = 512
_SUB = 128


@functools.lru_cache(maxsize=None)
def _build_sc_kernel(b_total: int, s_len: int, n_ctx: int, d: int,
                     nc: int, ns: int):
    nw = nc * ns
    bpb = b_total // _BBLK
    items = s_len * bpb
    items_pw = items // nw
    d_vecs = d // _LANES
    dt_n = d // 8
    bt_n = _BBLK // _SUB

    mesh = plsc.VectorSubcoreMesh(core_axis_name="c", subcore_axis_name="s")

    @functools.partial(
        pl.kernel,
        out_type=jax.ShapeDtypeStruct((s_len, dt_n, b_total // _SUB, 8, _SUB),
                                      jnp.float32),
        mesh=mesh,
        scratch_types=[
            pltpu.VMEM((n_ctx, d), jnp.float32),
            pltpu.VMEM((2, d), jnp.float32),
            pltpu.VMEM((2 * n_ctx * (d + 8),), jnp.float32),
            [pltpu.VMEM((_BBLK,), jnp.int32)] * _NBUF,
            [pltpu.VMEM((_BBLK,), jnp.int32)] * _NBUF,
            [pltpu.VMEM((dt_n, bt_n, 8, _SUB), jnp.float32)] * _NBUF,
            [pltpu.SemaphoreType.DMA] * _NBUF,
            [pltpu.SemaphoreType.DMA] * _NBUF,
        ],
        compiler_params=pltpu.CompilerParams(use_tc_tiling_on_sc=False,
                                             needs_layout_passes=False),
    )
    def sc_kernel(pos_hbm, seq_hbm, ptab_hbm, stab_hbm, out_hbm,
                  ptab_v, stab_v, comb, posv, seqv, outv, sem_in, sem_w):
        c = lax.axis_index("c")
        s = lax.axis_index("s")
        wid = c * ns + s
        item0 = wid * items_pw

        pltpu.sync_copy(ptab_hbm, ptab_v)
        pltpu.sync_copy(stab_hbm, stab_v)

        dp = d + 8

        def row(p, carry):
            for dc in range(d_vecs):
                sl = pl.ds(dc * _LANES, _LANES)
                v = ptab_v[p, sl]
                comb[pl.ds(p * dp + dc * _LANES, _LANES)] = v + stab_v[0, sl]
                comb[pl.ds(n_ctx * dp + p * dp + dc * _LANES, _LANES)] = (
                    v + stab_v[1, sl])
            return carry

        lax.fori_loop(0, n_ctx, row, 0)

        def load_idx(item, b):
            t = item // bpb
            b0 = (item % bpb) * _BBLK
            pltpu.async_copy(pos_hbm.at[t, pl.ds(b0, _BBLK)], posv[b],
                             sem_in[b])
            pltpu.async_copy(seq_hbm.at[t, pl.ds(b0, _BBLK)], seqv[b],
                             sem_in[b])

        def drain_idx(item, b):
            t = item // bpb
            b0 = (item % bpb) * _BBLK
            pltpu.make_async_copy(pos_hbm.at[t, pl.ds(b0, _BBLK)], posv[b],
                                  sem_in[b]).wait()
            pltpu.make_async_copy(seq_hbm.at[t, pl.ds(b0, _BBLK)], seqv[b],
                                  sem_in[b]).wait()

        def out_slice(item):
            t = item // bpb
            bb = item % bpb
            return out_hbm.at[t, :, pl.ds(bb * bt_n, bt_n), :, :]

        for b in range(_NBUF):
            load_idx(item0 + b, b)

        def super_body(m2, carry):
            for b in range(_NBUF):
                m = m2 * _NBUF + b
                item = item0 + m
                drain_idx(item, b)

                @pl.when(m2 > 0)
                def _drain_wb():
                    pltpu.make_async_copy(outv[b], out_slice(item),
                                          sem_w[b]).wait()

                @plsc.parallel_loop(0, _BBLK // _LANES)
                def jgroup(j):
                    bt = j // (_SUB // _LANES)
                    bcg = j % (_SUB // _LANES)
                    sl = pl.ds(j * _LANES, _LANES)
                    ci = seqv[b][sl] * (n_ctx * dp) + posv[b][sl] * dp
                    osl = pl.ds(bcg * _LANES, _LANES)
                    for dt in range(d // 8):
                        vs = [plsc.load_gather(comb, [ci + dt * 8 + dr])
                              for dr in range(8)]
                        for dr in range(8):
                            outv[b][dt, bt, dr, osl] = vs[dr]

                @pl.when(m + _NBUF < items_pw)
                def _prefetch():
                    load_idx(item + _NBUF, b)

                pltpu.async_copy(outv[b], out_slice(item), sem_w[b])
            return carry

        lax.fori_loop(0, items_pw // _NBUF, super_body, 0)

        for b in range(_NBUF):
            pltpu.make_async_copy(
                outv[b], out_slice(item0 + items_pw - _NBUF + b),
                sem_w[b]).wait()

    return sc_kernel


def kernel(positions, sequence_ids, pos_table, seq_table):
    b, s = positions.shape
    n_ctx, d = pos_table.shape

    info = plsc.get_sparse_core_info()
    nc, ns = info.num_cores, info.num_subcores
    nw = nc * ns

    assert (s * (b // _BBLK)) % (nw * _NBUF) == 0
    assert b % _BBLK == 0 and d % _LANES == 0

    sc = _build_sc_kernel(b, s, n_ctx, d, nc, ns)
    out5 = sc(positions.T.astype(jnp.int32), sequence_ids.T.astype(jnp.int32),
              pos_table.astype(jnp.float32), seq_table.astype(jnp.float32))
    return jnp.transpose(out5, (2, 4, 0, 1, 3)).reshape(b, s, d)


# speedup vs baseline: 8.1839x; 1.1005x over previous
import functools

import jax
import jax.numpy as jnp
from jax import lax
from jax.experimental import pallas as pl
from jax.experimental.pallas import tpu as pltpu
from jax.experimental.pallas import tpu_sc as plsc

_LANES = 16
_NBUF = 2
_BBLK = 512
_SUB = 128


@functools.lru_cache(maxsize=None)
def _build_sc_kernel(b_total: int, s_len: int, n_ctx: int, d: int,
                     nc: int, ns: int):
    nw = nc * ns
    bpb = b_total // _BBLK
    items = s_len * bpb
    items_pw = items // nw
    d_vecs = d // _LANES
    dt_n = d // 8
    bt_n = _BBLK // _SUB

    mesh = plsc.VectorSubcoreMesh(core_axis_name="c", subcore_axis_name="s")

    @functools.partial(
        pl.kernel,
        out_type=jax.ShapeDtypeStruct((s_len, dt_n, b_total // _SUB, 8, _SUB),
                                      jnp.float32),
        mesh=mesh,
        scratch_types=[
            pltpu.VMEM((n_ctx, d), jnp.float32),
            pltpu.VMEM((2, d), jnp.float32),
            pltpu.VMEM((2 * n_ctx * (d + 8),), jnp.float32),
            [pltpu.VMEM((_BBLK,), jnp.int32)] * _NBUF,
            [pltpu.VMEM((_BBLK,), jnp.int32)] * _NBUF,
            [pltpu.VMEM((dt_n, bt_n, 8, _SUB), jnp.float32)] * _NBUF,
            [pltpu.SemaphoreType.DMA] * _NBUF,
            [pltpu.SemaphoreType.DMA] * _NBUF,
        ],
        compiler_params=pltpu.CompilerParams(use_tc_tiling_on_sc=False,
                                             needs_layout_passes=False),
    )
    def sc_kernel(pos_hbm, seq_hbm, ptab_hbm, stab_hbm, out_hbm,
                  ptab_v, stab_v, comb, posv, seqv, outv, sem_in, sem_w):
        c = lax.axis_index("c")
        s = lax.axis_index("s")
        wid = c * ns + s
        item0 = wid * items_pw

        pltpu.sync_copy(ptab_hbm, ptab_v)
        pltpu.sync_copy(stab_hbm, stab_v)

        dp = d + 8

        def row(p, carry):
            for dc in range(d_vecs):
                sl = pl.ds(dc * _LANES, _LANES)
                v = ptab_v[p, sl]
                comb[pl.ds(p * dp + dc * _LANES, _LANES)] = v + stab_v[0, sl]
                comb[pl.ds(n_ctx * dp + p * dp + dc * _LANES, _LANES)] = (
                    v + stab_v[1, sl])
            return carry

        lax.fori_loop(0, n_ctx, row, 0)

        def load_idx(item, b):
            t = item // bpb
            b0 = (item % bpb) * _BBLK
            pltpu.async_copy(pos_hbm.at[t, pl.ds(b0, _BBLK)], posv[b],
                             sem_in[b])
            pltpu.async_copy(seq_hbm.at[t, pl.ds(b0, _BBLK)], seqv[b],
                             sem_in[b])

        def drain_idx(item, b):
            t = item // bpb
            b0 = (item % bpb) * _BBLK
            pltpu.make_async_copy(pos_hbm.at[t, pl.ds(b0, _BBLK)], posv[b],
                                  sem_in[b]).wait()
            pltpu.make_async_copy(seq_hbm.at[t, pl.ds(b0, _BBLK)], seqv[b],
                                  sem_in[b]).wait()

        def out_slice(item):
            t = item // bpb
            bb = item % bpb
            return out_hbm.at[t, :, pl.ds(bb * bt_n, bt_n), :, :]

        for b in range(_NBUF):
            load_idx(item0 + b, b)

        def super_body(m2, carry):
            for b in range(_NBUF):
                m = m2 * _NBUF + b
                item = item0 + m
                drain_idx(item, b)

                @pl.when(m2 > 0)
                def _drain_wb():
                    pltpu.make_async_copy(outv[b], out_slice(item),
                                          sem_w[b]).wait()

                @plsc.parallel_loop(0, _BBLK // _LANES)
                def jgroup(j):
                    bt = j // (_SUB // _LANES)
                    bcg = j % (_SUB // _LANES)
                    sl = pl.ds(j * _LANES, _LANES)
                    ci = seqv[b][sl] * (n_ctx * dp) + posv[b][sl] * dp
                    osl = pl.ds(bcg * _LANES, _LANES)
                    for dg in range(d // 16):
                        vs = [plsc.load_gather(comb, [ci + dg * 16 + dr])
                              for dr in range(16)]
                        for dr in range(16):
                            dd = dg * 16 + dr
                            outv[b][dd // 8, bt, dd % 8, osl] = vs[dr]

                @pl.when(m + _NBUF < items_pw)
                def _prefetch():
                    load_idx(item + _NBUF, b)

                pltpu.async_copy(outv[b], out_slice(item), sem_w[b])
            return carry

        lax.fori_loop(0, items_pw // _NBUF, super_body, 0)

        for b in range(_NBUF):
            pltpu.make_async_copy(
                outv[b], out_slice(item0 + items_pw - _NBUF + b),
                sem_w[b]).wait()

    return sc_kernel


def kernel(positions, sequence_ids, pos_table, seq_table):
    b, s = positions.shape
    n_ctx, d = pos_table.shape

    info = plsc.get_sparse_core_info()
    nc, ns = info.num_cores, info.num_subcores
    nw = nc * ns

    assert (s * (b // _BBLK)) % (nw * _NBUF) == 0
    assert b % _BBLK == 0 and d % _LANES == 0

    sc = _build_sc_kernel(b, s, n_ctx, d, nc, ns)
    out5 = sc(positions.T.astype(jnp.int32), sequence_ids.T.astype(jnp.int32),
              pos_table.astype(jnp.float32), seq_table.astype(jnp.float32))
    return jnp.transpose(out5, (2, 4, 0, 1, 3)).reshape(b, s, d)
